# Initial kernel scaffold; baseline (speedup 1.0000x reference)
import jax
import jax.numpy as jnp
from jax.experimental import pallas as pl


def kernel(x, edge_index, W1, b1, W2, b2, Wfc, bfc):
    raise NotImplementedError("write your pallas kernel here")


# speedup vs baseline: 12.6885x; 12.6885x over previous
import functools

import jax
import jax.numpy as jnp
from jax import lax
from jax.experimental import pallas as pl
from jax.experimental.pallas import tpu as pltpu
from jax.experimental.pallas import tpu_sc as plsc

N = 10000
E = 320000
F = 128
NCLASS = 40

NC = 2
NS = 16
NW = NC * NS
EW = E // NW
BE = 80
NB = EW // BE
RPT = 624
TAIL = N - NS * RPT


def _acc_zero(zeros_hbm, acc_sh, s):
    pltpu.sync_copy(zeros_hbm.at[pl.ds(s * RPT, RPT)], acc_sh.at[pl.ds(s * RPT, RPT)])

    @pl.when(s == NS - 1)
    def _():
        pltpu.sync_copy(zeros_hbm.at[pl.ds(NS * RPT, TAIL)],
                        acc_sh.at[pl.ds(NS * RPT, TAIL)])


def _acc_spill(acc_sh, out_hbm, c, s):
    pltpu.sync_copy(acc_sh.at[pl.ds(s * RPT, RPT)], out_hbm.at[c, pl.ds(s * RPT, RPT)])

    @pl.when(s == NS - 1)
    def _():
        pltpu.sync_copy(acc_sh.at[pl.ds(NS * RPT, TAIL)],
                        out_hbm.at[c, pl.ds(NS * RPT, TAIL)])

_mesh = plsc.VectorSubcoreMesh(
    core_axis_name="c", subcore_axis_name="s", num_cores=NC, num_subcores=NS
)


@functools.partial(
    pl.kernel,
    out_type=jax.ShapeDtypeStruct((NC, N, 16), jnp.float32),
    mesh=_mesh,
    scratch_types=[
        pltpu.VMEM((BE,), jnp.int32),
        pltpu.VMEM((BE, 16), jnp.float32),
        pltpu.VMEM_SHARED((N, 16), jnp.float32),
    ],
)
def _deg_kernel(dst_hbm, zeros_hbm, ones_hbm, out_hbm, idx_v, ones_v, acc_sh):
    c = lax.axis_index("c")
    s = lax.axis_index("s")
    w = s * NC + c
    _acc_zero(zeros_hbm, acc_sh, s)
    pltpu.sync_copy(ones_hbm, ones_v)
    plsc.subcore_barrier()

    def body(i, carry):
        base = w * EW + i * BE
        pltpu.sync_copy(dst_hbm.at[pl.ds(base, BE)], idx_v)
        pltpu.sync_copy(ones_v, acc_sh.at[idx_v], add=True)
        return carry

    lax.fori_loop(0, NB, body, 0)
    plsc.subcore_barrier()
    _acc_spill(acc_sh, out_hbm, c, s)


@functools.partial(
    pl.kernel,
    out_type=jax.ShapeDtypeStruct((NC, N, F), jnp.float32),
    mesh=_mesh,
    scratch_types=[
        pltpu.VMEM((BE,), jnp.int32),
        pltpu.VMEM((BE,), jnp.int32),
        pltpu.VMEM((BE, F), jnp.float32),
        pltpu.VMEM_SHARED((N, F), jnp.float32),
        pltpu.SemaphoreType.DMA,
    ],
)
def _agg_kernel(g_hbm, src_hbm, dst_hbm, zeros_hbm, out_hbm,
                src_v, dst_v, rows_v, acc_sh, sem):
    c = lax.axis_index("c")
    s = lax.axis_index("s")
    w = s * NC + c
    _acc_zero(zeros_hbm, acc_sh, s)
    plsc.subcore_barrier()

    def body(i, carry):
        base = w * EW + i * BE
        pltpu.sync_copy(src_hbm.at[pl.ds(base, BE)], src_v)
        pltpu.sync_copy(dst_hbm.at[pl.ds(base, BE)], dst_v)
        pltpu.async_copy(g_hbm.at[src_v], rows_v, sem).wait()
        pltpu.sync_copy(rows_v, acc_sh.at[dst_v], add=True)
        return carry

    lax.fori_loop(0, NB, body, 0)
    plsc.subcore_barrier()
    _acc_spill(acc_sh, out_hbm, c, s)


BR = 1000


def _dis(degp_ref):
    deg = degp_ref[0, :, 0:1] + degp_ref[1, :, 0:1] + 1.0
    return lax.rsqrt(deg)


def _k1_body(x_ref, w_ref, degp_ref, o_ref):
    dis = _dis(degp_ref)
    o_ref[...] = dis * jnp.dot(x_ref[...], w_ref[...],
                               preferred_element_type=jnp.float32)


def _k2_body(p_ref, g_ref, b_ref, degp_ref, w_ref, h_ref, g2_ref):
    dis = _dis(degp_ref)
    h = jnp.maximum(dis * (p_ref[0] + p_ref[1] + g_ref[...]) + b_ref[...], 0.0)
    h_ref[...] = h
    g2_ref[...] = dis * jnp.dot(h, w_ref[...],
                                preferred_element_type=jnp.float32)


def _k3_body(p_ref, g_ref, b_ref, degp_ref, h1_ref, wfc_ref, bfc_ref,
             emb_ref, log_ref):
    dis = _dis(degp_ref)
    h2 = jnp.maximum(dis * (p_ref[0] + p_ref[1] + g_ref[...]) + b_ref[...], 0.0)
    emb = jnp.maximum(h1_ref[...], h2)
    emb_ref[...] = emb
    log_ref[...] = jnp.dot(emb, wfc_ref[...],
                           preferred_element_type=jnp.float32) + bfc_ref[...]


def _row_spec(width):
    return pl.BlockSpec((BR, width), lambda i: (i, 0))


def _part_spec(width):
    return pl.BlockSpec((NC, BR, width), lambda i: (0, i, 0))


def _full_spec(shape):
    return pl.BlockSpec(shape, lambda i: tuple(0 for _ in shape))


def kernel(x, edge_index, W1, b1, W2, b2, Wfc, bfc):
    src = edge_index[0].astype(jnp.int32)
    dst = edge_index[1].astype(jnp.int32)
    z16 = jnp.zeros((N, 16), jnp.float32)
    zF = jnp.zeros((N, F), jnp.float32)
    ones = jnp.ones((BE, 16), jnp.float32)

    degp = _deg_kernel(dst, z16, ones)

    g1 = pl.pallas_call(
        _k1_body,
        grid=(N // BR,),
        in_specs=[_row_spec(F), _full_spec((F, F)), _part_spec(16)],
        out_specs=_row_spec(F),
        out_shape=jax.ShapeDtypeStruct((N, F), jnp.float32),
    )(x, W1, degp)

    p1 = _agg_kernel(g1, src, dst, zF)

    h1, g2 = pl.pallas_call(
        _k2_body,
        grid=(N // BR,),
        in_specs=[_part_spec(F), _row_spec(F), _full_spec((1, F)),
                  _part_spec(16), _full_spec((F, F))],
        out_specs=[_row_spec(F), _row_spec(F)],
        out_shape=[jax.ShapeDtypeStruct((N, F), jnp.float32),
                   jax.ShapeDtypeStruct((N, F), jnp.float32)],
    )(p1, g1, b1.reshape(1, F), degp, W2)

    p2 = _agg_kernel(g2, src, dst, zF)

    emb, logits = pl.pallas_call(
        _k3_body,
        grid=(N // BR,),
        in_specs=[_part_spec(F), _row_spec(F), _full_spec((1, F)),
                  _part_spec(16), _row_spec(F), _full_spec((F, NCLASS)),
                  _full_spec((1, NCLASS))],
        out_specs=[_row_spec(F), _row_spec(NCLASS)],
        out_shape=[jax.ShapeDtypeStruct((N, F), jnp.float32),
                   jax.ShapeDtypeStruct((N, NCLASS), jnp.float32)],
    )(p2, g2, b2.reshape(1, F), degp, h1, Wfc, bfc.reshape(1, NCLASS))

    return emb, logits


# speedup vs baseline: 21.5338x; 1.6971x over previous
import functools

import jax
import jax.numpy as jnp
from jax import lax
from jax.experimental import pallas as pl
from jax.experimental.pallas import tpu as pltpu
from jax.experimental.pallas import tpu_sc as plsc

N = 10000
E = 320000
F = 128
NCLASS = 40

NC = 2
NS = 16
NW = NC * NS
EW = E // NW
BE = 80
NB = EW // BE
NBUF = 2
NO = NB // NBUF

RPT = 624
TAIL = N - NS * RPT

_mesh = plsc.VectorSubcoreMesh(
    core_axis_name="c", subcore_axis_name="s", num_cores=NC, num_subcores=NS
)


def _acc_zero(zeros_hbm, acc_sh, s):
    pltpu.sync_copy(zeros_hbm.at[pl.ds(s * RPT, RPT)], acc_sh.at[pl.ds(s * RPT, RPT)])

    @pl.when(s == NS - 1)
    def _():
        pltpu.sync_copy(zeros_hbm.at[pl.ds(NS * RPT, TAIL)],
                        acc_sh.at[pl.ds(NS * RPT, TAIL)])


def _acc_spill(acc_sh, out_hbm, c, s):
    pltpu.sync_copy(acc_sh.at[pl.ds(s * RPT, RPT)], out_hbm.at[c, pl.ds(s * RPT, RPT)])

    @pl.when(s == NS - 1)
    def _():
        pltpu.sync_copy(acc_sh.at[pl.ds(NS * RPT, TAIL)],
                        out_hbm.at[c, pl.ds(NS * RPT, TAIL)])


@functools.partial(
    pl.kernel,
    out_type=jax.ShapeDtypeStruct((NC, N, F), jnp.float32),
    mesh=_mesh,
    scratch_types=[
        pltpu.VMEM((NB, BE), jnp.int32),
        pltpu.VMEM((BE, F), jnp.float32),
        pltpu.VMEM_SHARED((N, F), jnp.float32),
        [pltpu.SemaphoreType.DMA] * NBUF,
    ],
)
def _deg_kernel(dst_hbm, zeros_hbm, ones_hbm, out_hbm, idx_v, ones_v, acc_sh, sems):
    c = lax.axis_index("c")
    s = lax.axis_index("s")
    w = s * NC + c
    _acc_zero(zeros_hbm, acc_sh, s)
    pltpu.sync_copy(dst_hbm.at[w], idx_v)
    pltpu.sync_copy(ones_hbm, ones_v)
    plsc.subcore_barrier()

    def scatter_start(i, b):
        pltpu.async_copy(ones_v, acc_sh.at[idx_v.at[i]], sems[b], add=True)

    def scatter_wait(b):
        pltpu.make_async_copy(ones_v, acc_sh.at[idx_v.at[0]], sems[b]).wait()

    for b in range(NBUF):
        scatter_start(b, b)

    def outer(o, carry):
        for b in range(NBUF):
            nxt = (o + 1) * NBUF + b
            scatter_wait(b)

            @pl.when(nxt < NB)
            def _():
                scatter_start(nxt, b)

        return carry

    lax.fori_loop(0, NB // NBUF, outer, 0)
    for t in range((NB // NBUF) * NBUF, NB):
        scatter_wait(t % NBUF)
    plsc.subcore_barrier()
    _acc_spill(acc_sh, out_hbm, c, s)


@functools.partial(
    pl.kernel,
    out_type=jax.ShapeDtypeStruct((NC, N, F), jnp.float32),
    mesh=_mesh,
    scratch_types=[
        pltpu.VMEM((EW,), jnp.int32),
        pltpu.VMEM((NB, BE), jnp.int32),
        [pltpu.VMEM((BE, F), jnp.float32)] * NBUF,
        [pltpu.SemaphoreType.DMA] * NBUF,
        [pltpu.SemaphoreType.DMA] * NBUF,
        pltpu.VMEM_SHARED((N, F), jnp.float32),
    ],
)
def _agg_kernel(g_hbm, src_hbm, dst_hbm, zeros_hbm, out_hbm,
                src_v, dst_v, rows, gsem, ssem, acc_sh):
    c = lax.axis_index("c")
    s = lax.axis_index("s")
    w = s * NC + c
    _acc_zero(zeros_hbm, acc_sh, s)
    pltpu.sync_copy(src_hbm.at[pl.ds(w * EW, EW)], src_v)
    pltpu.sync_copy(dst_hbm.at[w], dst_v)
    plsc.subcore_barrier()

    def gather_start(i, b):
        pltpu.async_copy(g_hbm.at[src_v.at[pl.ds(i * BE, BE)]], rows[b], gsem[b])

    def gather_wait(b):
        pltpu.make_async_copy(g_hbm.at[src_v.at[pl.ds(0, BE)]], rows[b],
                              gsem[b]).wait()

    def scatter_start(i, b):
        pltpu.async_copy(rows[b], acc_sh.at[dst_v.at[i]], ssem[b], add=True)

    def scatter_wait(b):
        pltpu.make_async_copy(rows[b], acc_sh.at[dst_v.at[0]], ssem[b]).wait()

    for b in range(NBUF):
        gather_start(b, b)

    def outer(o, carry):
        for b in range(NBUF):
            gather_wait(b)
            scatter_start(o * NBUF + b, b)
        for b in range(NBUF):
            nxt = (o + 1) * NBUF + b
            scatter_wait(b)

            @pl.when(nxt < NB)
            def _():
                gather_start(nxt, b)

        return carry

    lax.fori_loop(0, NO, outer, 0)
    for t in range(NO * NBUF, NB):
        b = t - NO * NBUF
        gather_wait(b)
        scatter_start(t, b)
        scatter_wait(b)
    plsc.subcore_barrier()
    _acc_spill(acc_sh, out_hbm, c, s)


BR = 1000


def _dis(degp_ref):
    deg = degp_ref[0, :, 0:1] + degp_ref[1, :, 0:1] + 1.0
    return lax.rsqrt(deg)


def _k1_body(x_ref, w_ref, degp_ref, o_ref):
    dis = _dis(degp_ref)
    o_ref[...] = dis * jnp.dot(x_ref[...], w_ref[...],
                               preferred_element_type=jnp.float32)


def _k2_body(p_ref, g_ref, b_ref, degp_ref, w_ref, h_ref, g2_ref):
    dis = _dis(degp_ref)
    h = jnp.maximum(dis * (p_ref[0] + p_ref[1] + g_ref[...]) + b_ref[...], 0.0)
    h_ref[...] = h
    g2_ref[...] = dis * jnp.dot(h, w_ref[...],
                                preferred_element_type=jnp.float32)


def _k3_body(p_ref, g_ref, b_ref, degp_ref, h1_ref, wfc_ref, bfc_ref,
             emb_ref, log_ref):
    dis = _dis(degp_ref)
    h2 = jnp.maximum(dis * (p_ref[0] + p_ref[1] + g_ref[...]) + b_ref[...], 0.0)
    emb = jnp.maximum(h1_ref[...], h2)
    emb_ref[...] = emb
    log_ref[...] = jnp.dot(emb, wfc_ref[...],
                           preferred_element_type=jnp.float32) + bfc_ref[...]


def _row_spec(width):
    return pl.BlockSpec((BR, width), lambda i: (i, 0))


def _part_spec(width):
    return pl.BlockSpec((NC, BR, width), lambda i: (0, i, 0))


def _full_spec(shape):
    return pl.BlockSpec(shape, lambda i: tuple(0 for _ in shape))


def kernel(x, edge_index, W1, b1, W2, b2, Wfc, bfc):
    src = edge_index[0].astype(jnp.int32)
    dst = edge_index[1].astype(jnp.int32).reshape(NW, NB, BE)
    zF = jnp.zeros((N, F), jnp.float32)
    ones = jnp.ones((BE, F), jnp.float32)

    degp = _deg_kernel(dst, zF, ones)

    g1 = pl.pallas_call(
        _k1_body,
        grid=(N // BR,),
        in_specs=[_row_spec(F), _full_spec((F, F)), _part_spec(F)],
        out_specs=_row_spec(F),
        out_shape=jax.ShapeDtypeStruct((N, F), jnp.float32),
    )(x, W1, degp)

    p1 = _agg_kernel(g1, src, dst, zF)

    h1, g2 = pl.pallas_call(
        _k2_body,
        grid=(N // BR,),
        in_specs=[_part_spec(F), _row_spec(F), _full_spec((1, F)),
                  _part_spec(F), _full_spec((F, F))],
        out_specs=[_row_spec(F), _row_spec(F)],
        out_shape=[jax.ShapeDtypeStruct((N, F), jnp.float32),
                   jax.ShapeDtypeStruct((N, F), jnp.float32)],
    )(p1, g1, b1.reshape(1, F), degp, W2)

    p2 = _agg_kernel(g2, src, dst, zF)

    emb, logits = pl.pallas_call(
        _k3_body,
        grid=(N // BR,),
        in_specs=[_part_spec(F), _row_spec(F), _full_spec((1, F)),
                  _part_spec(F), _row_spec(F), _full_spec((F, NCLASS)),
                  _full_spec((1, NCLASS))],
        out_specs=[_row_spec(F), _row_spec(NCLASS)],
        out_shape=[jax.ShapeDtypeStruct((N, F), jnp.float32),
                   jax.ShapeDtypeStruct((N, NCLASS), jnp.float32)],
    )(p2, g2, b2.reshape(1, F), degp, h1, Wfc, bfc.reshape(1, NCLASS))

    return emb, logits


# speedup vs baseline: 24.8786x; 1.1553x over previous
import functools

import jax
import jax.numpy as jnp
from jax import lax
from jax.experimental import pallas as pl
from jax.experimental.pallas import tpu as pltpu
from jax.experimental.pallas import tpu_sc as plsc

N = 10000
E = 320000
F = 128
NCLASS = 40

NC = 2
NS = 16
NW = NC * NS
EW = E // NW
BE = 80
NB = EW // BE
NBUF = 2
NO = NB // NBUF

RPT = 624
TAIL = N - NS * RPT

_mesh = plsc.VectorSubcoreMesh(
    core_axis_name="c", subcore_axis_name="s", num_cores=NC, num_subcores=NS
)


def _acc_zero(zeros_hbm, acc_sh, s):
    pltpu.sync_copy(zeros_hbm.at[pl.ds(s * RPT, RPT)], acc_sh.at[pl.ds(s * RPT, RPT)])

    @pl.when(s == NS - 1)
    def _():
        pltpu.sync_copy(zeros_hbm.at[pl.ds(NS * RPT, TAIL)],
                        acc_sh.at[pl.ds(NS * RPT, TAIL)])


def _acc_spill(acc_sh, out_hbm, c, s):
    pltpu.sync_copy(acc_sh.at[pl.ds(s * RPT, RPT)], out_hbm.at[c, pl.ds(s * RPT, RPT)])

    @pl.when(s == NS - 1)
    def _():
        pltpu.sync_copy(acc_sh.at[pl.ds(NS * RPT, TAIL)],
                        out_hbm.at[c, pl.ds(NS * RPT, TAIL)])


@functools.partial(
    pl.kernel,
    out_type=jax.ShapeDtypeStruct((NC, N, F), jnp.float32),
    mesh=_mesh,
    scratch_types=[
        pltpu.VMEM((NB, BE), jnp.int32),
        pltpu.VMEM((BE, F), jnp.float32),
        pltpu.VMEM_SHARED((N, F), jnp.float32),
        [pltpu.SemaphoreType.DMA] * NBUF,
    ],
)
def _deg_kernel(dst_hbm, zeros_hbm, ones_hbm, out_hbm, idx_v, ones_v, acc_sh, sems):
    c = lax.axis_index("c")
    s = lax.axis_index("s")
    w = s * NC + c
    _acc_zero(zeros_hbm, acc_sh, s)
    pltpu.sync_copy(dst_hbm.at[w], idx_v)
    pltpu.sync_copy(ones_hbm, ones_v)
    plsc.subcore_barrier()

    def scatter_start(i, b):
        pltpu.async_copy(ones_v, acc_sh.at[idx_v.at[i]], sems[b], add=True)

    def scatter_wait(b):
        pltpu.make_async_copy(ones_v, acc_sh.at[idx_v.at[0]], sems[b]).wait()

    for b in range(NBUF):
        scatter_start(b, b)

    def outer(o, carry):
        for b in range(NBUF):
            nxt = (o + 1) * NBUF + b
            scatter_wait(b)

            @pl.when(nxt < NB)
            def _():
                scatter_start(nxt, b)

        return carry

    lax.fori_loop(0, NB // NBUF, outer, 0)
    for t in range((NB // NBUF) * NBUF, NB):
        scatter_wait(t % NBUF)
    plsc.subcore_barrier()
    _acc_spill(acc_sh, out_hbm, c, s)


NBA = 3


@functools.partial(
    pl.kernel,
    out_type=jax.ShapeDtypeStruct((NC, N, F), jnp.float32),
    mesh=_mesh,
    scratch_types=[
        pltpu.VMEM((NBA, BE), jnp.int32),
        pltpu.VMEM((NB, BE), jnp.int32),
        [pltpu.VMEM((BE, F), jnp.float32)] * NBA,
        [pltpu.SemaphoreType.DMA] * NBA,
        [pltpu.SemaphoreType.DMA] * NBA,
        [pltpu.SemaphoreType.DMA] * NBA,
        pltpu.VMEM_SHARED((N, F), jnp.float32),
    ],
)
def _agg_kernel(g_hbm, src_hbm, dst_hbm, zeros_hbm, out_hbm,
                src_v, dst_v, rows, isem, gsem, ssem, acc_sh):
    c = lax.axis_index("c")
    s = lax.axis_index("s")
    w = s * NC + c
    _acc_zero(zeros_hbm, acc_sh, s)
    pltpu.sync_copy(dst_hbm.at[w], dst_v)
    plsc.subcore_barrier()

    def idx_start(i, b):
        pltpu.async_copy(src_hbm.at[pl.ds(w * EW + i * BE, BE)], src_v.at[b],
                         isem[b])

    def idx_wait(b):
        pltpu.make_async_copy(src_hbm.at[pl.ds(0, BE)], src_v.at[b],
                              isem[b]).wait()

    def gather_start(b):
        pltpu.async_copy(g_hbm.at[src_v.at[b]], rows[b], gsem[b])

    def gather_wait(b):
        pltpu.make_async_copy(g_hbm.at[src_v.at[0]], rows[b], gsem[b]).wait()

    def scatter_start(i, b):
        pltpu.async_copy(rows[b], acc_sh.at[dst_v.at[i]], ssem[b], add=True)

    def scatter_wait(b):
        pltpu.make_async_copy(rows[b], acc_sh.at[dst_v.at[0]], ssem[b]).wait()

    for b in range(NBA):
        idx_start(b, b)
    for b in range(NBA):
        idx_wait(b)
        gather_start(b)

    def outer(o, carry):
        for b in range(NBA):
            i = o * NBA + b
            gather_wait(b)
            scatter_start(i, b)

            @pl.when(i + NBA < NB)
            def _():
                idx_start(i + NBA, b)

        for b in range(NBA):
            nxt = (o + 1) * NBA + b
            scatter_wait(b)

            @pl.when(nxt < NB)
            def _():
                idx_wait(b)
                gather_start(b)

        return carry

    lax.fori_loop(0, NB // NBA, outer, 0)
    for t in range((NB // NBA) * NBA, NB):
        b = t % NBA
        gather_wait(b)
        scatter_start(t, b)
        scatter_wait(b)
    plsc.subcore_barrier()
    _acc_spill(acc_sh, out_hbm, c, s)


BR = 1000


def _dis(degp_ref):
    deg = degp_ref[0, :, 0:1] + degp_ref[1, :, 0:1] + 1.0
    return lax.rsqrt(deg)


def _k1_body(x_ref, w_ref, degp_ref, o_ref):
    dis = _dis(degp_ref)
    o_ref[...] = dis * jnp.dot(x_ref[...], w_ref[...],
                               preferred_element_type=jnp.float32)


def _k2_body(p_ref, g_ref, b_ref, degp_ref, w_ref, h_ref, g2_ref):
    dis = _dis(degp_ref)
    h = jnp.maximum(dis * (p_ref[0] + p_ref[1] + g_ref[...]) + b_ref[...], 0.0)
    h_ref[...] = h
    g2_ref[...] = dis * jnp.dot(h, w_ref[...],
                                preferred_element_type=jnp.float32)


def _k3_body(p_ref, g_ref, b_ref, degp_ref, h1_ref, wfc_ref, bfc_ref,
             emb_ref, log_ref):
    dis = _dis(degp_ref)
    h2 = jnp.maximum(dis * (p_ref[0] + p_ref[1] + g_ref[...]) + b_ref[...], 0.0)
    emb = jnp.maximum(h1_ref[...], h2)
    emb_ref[...] = emb
    log_ref[...] = jnp.dot(emb, wfc_ref[...],
                           preferred_element_type=jnp.float32) + bfc_ref[...]


def _row_spec(width):
    return pl.BlockSpec((BR, width), lambda i: (i, 0))


def _part_spec(width):
    return pl.BlockSpec((NC, BR, width), lambda i: (0, i, 0))


def _full_spec(shape):
    return pl.BlockSpec(shape, lambda i: tuple(0 for _ in shape))


def kernel(x, edge_index, W1, b1, W2, b2, Wfc, bfc):
    src = edge_index[0].astype(jnp.int32)
    dst = edge_index[1].astype(jnp.int32).reshape(NW, NB, BE)
    zF = jnp.zeros((N, F), jnp.float32)
    ones = jnp.ones((BE, F), jnp.float32)

    degp = _deg_kernel(dst, zF, ones)[:, :, :8]

    g1 = pl.pallas_call(
        _k1_body,
        grid=(N // BR,),
        in_specs=[_row_spec(F), _full_spec((F, F)), _part_spec(8)],
        out_specs=_row_spec(F),
        out_shape=jax.ShapeDtypeStruct((N, F), jnp.float32),
    )(x, W1, degp)

    p1 = _agg_kernel(g1, src, dst, zF)

    h1, g2 = pl.pallas_call(
        _k2_body,
        grid=(N // BR,),
        in_specs=[_part_spec(F), _row_spec(F), _full_spec((1, F)),
                  _part_spec(8), _full_spec((F, F))],
        out_specs=[_row_spec(F), _row_spec(F)],
        out_shape=[jax.ShapeDtypeStruct((N, F), jnp.float32),
                   jax.ShapeDtypeStruct((N, F), jnp.float32)],
    )(p1, g1, b1.reshape(1, F), degp, W2)

    p2 = _agg_kernel(g2, src, dst, zF)

    emb, logits = pl.pallas_call(
        _k3_body,
        grid=(N // BR,),
        in_specs=[_part_spec(F), _row_spec(F), _full_spec((1, F)),
                  _part_spec(8), _row_spec(F), _full_spec((F, NCLASS)),
                  _full_spec((1, NCLASS))],
        out_specs=[_row_spec(F), _row_spec(NCLASS)],
        out_shape=[jax.ShapeDtypeStruct((N, F), jnp.float32),
                   jax.ShapeDtypeStruct((N, NCLASS), jnp.float32)],
    )(p2, g2, b2.reshape(1, F), degp, h1, Wfc, bfc.reshape(1, NCLASS))

    return emb, logits


# speedup vs baseline: 26.2723x; 1.0560x over previous
import functools

import jax
import jax.numpy as jnp
from jax import lax
from jax.experimental import pallas as pl
from jax.experimental.pallas import tpu as pltpu
from jax.experimental.pallas import tpu_sc as plsc

N = 10000
E = 320000
F = 128
NCLASS = 40

NC = 2
NS = 16
NW = NC * NS
EW = E // NW
BE = 80
NB = EW // BE
NBUF = 2
NO = NB // NBUF

RPT = 624
TAIL = N - NS * RPT

_mesh = plsc.VectorSubcoreMesh(
    core_axis_name="c", subcore_axis_name="s", num_cores=NC, num_subcores=NS
)


def _acc_zero(zeros_hbm, acc_sh, s):
    pltpu.sync_copy(zeros_hbm.at[pl.ds(s * RPT, RPT)], acc_sh.at[pl.ds(s * RPT, RPT)])

    @pl.when(s == NS - 1)
    def _():
        pltpu.sync_copy(zeros_hbm.at[pl.ds(NS * RPT, TAIL)],
                        acc_sh.at[pl.ds(NS * RPT, TAIL)])


def _acc_spill(acc_sh, out_hbm, c, s):
    pltpu.sync_copy(acc_sh.at[pl.ds(s * RPT, RPT)], out_hbm.at[c, pl.ds(s * RPT, RPT)])

    @pl.when(s == NS - 1)
    def _():
        pltpu.sync_copy(acc_sh.at[pl.ds(NS * RPT, TAIL)],
                        out_hbm.at[c, pl.ds(NS * RPT, TAIL)])


@functools.partial(
    pl.kernel,
    out_type=jax.ShapeDtypeStruct((NC, N, F), jnp.float32),
    mesh=_mesh,
    scratch_types=[
        pltpu.VMEM((NB, BE), jnp.int32),
        pltpu.VMEM((BE, F), jnp.float32),
        pltpu.VMEM_SHARED((N, F), jnp.float32),
        [pltpu.SemaphoreType.DMA] * NBUF,
    ],
)
def _deg_kernel(dst_hbm, zeros_hbm, ones_hbm, out_hbm, idx_v, ones_v, acc_sh, sems):
    c = lax.axis_index("c")
    s = lax.axis_index("s")
    w = s * NC + c
    _acc_zero(zeros_hbm, acc_sh, s)
    pltpu.sync_copy(dst_hbm.at[w], idx_v)
    pltpu.sync_copy(ones_hbm, ones_v)
    plsc.subcore_barrier()

    def scatter_start(i, b):
        pltpu.async_copy(ones_v, acc_sh.at[idx_v.at[i]], sems[b], add=True)

    def scatter_wait(b):
        pltpu.make_async_copy(ones_v, acc_sh.at[idx_v.at[0]], sems[b]).wait()

    for b in range(NBUF):
        scatter_start(b, b)

    def outer(o, carry):
        for b in range(NBUF):
            nxt = (o + 1) * NBUF + b
            scatter_wait(b)

            @pl.when(nxt < NB)
            def _():
                scatter_start(nxt, b)

        return carry

    lax.fori_loop(0, NB // NBUF, outer, 0)
    for t in range((NB // NBUF) * NBUF, NB):
        scatter_wait(t % NBUF)
    plsc.subcore_barrier()
    _acc_spill(acc_sh, out_hbm, c, s)


NBA = 4


@functools.partial(
    pl.kernel,
    out_type=jax.ShapeDtypeStruct((NC, N, F), jnp.float32),
    mesh=_mesh,
    scratch_types=[
        pltpu.VMEM((NBA, BE), jnp.int32),
        pltpu.VMEM((NBA, BE), jnp.int32),
        [pltpu.VMEM((BE, F), jnp.float32)] * NBA,
        [pltpu.SemaphoreType.DMA] * NBA,
        [pltpu.SemaphoreType.DMA] * NBA,
        [pltpu.SemaphoreType.DMA] * NBA,
        [pltpu.SemaphoreType.DMA] * NBA,
        pltpu.VMEM_SHARED((N, F), jnp.float32),
    ],
)
def _agg_kernel(g_hbm, src_hbm, dst_hbm, zeros_hbm, out_hbm,
                src_v, dst_v, rows, isem, dsem, gsem, ssem, acc_sh):
    c = lax.axis_index("c")
    s = lax.axis_index("s")
    w = s * NC + c
    _acc_zero(zeros_hbm, acc_sh, s)
    plsc.subcore_barrier()

    def sidx_start(i, b):
        pltpu.async_copy(src_hbm.at[pl.ds(w * EW + i * BE, BE)], src_v.at[b],
                         isem[b])

    def sidx_wait(b):
        pltpu.make_async_copy(src_hbm.at[pl.ds(0, BE)], src_v.at[b],
                              isem[b]).wait()

    def didx_start(i, b):
        pltpu.async_copy(dst_hbm.at[pl.ds(w * EW + i * BE, BE)], dst_v.at[b],
                         dsem[b])

    def didx_wait(b):
        pltpu.make_async_copy(dst_hbm.at[pl.ds(0, BE)], dst_v.at[b],
                              dsem[b]).wait()

    def gather_start(b):
        pltpu.async_copy(g_hbm.at[src_v.at[b]], rows[b], gsem[b])

    def gather_wait(b):
        pltpu.make_async_copy(g_hbm.at[src_v.at[0]], rows[b], gsem[b]).wait()

    def scatter_start(b):
        pltpu.async_copy(rows[b], acc_sh.at[dst_v.at[b]], ssem[b], add=True)

    def scatter_wait(b):
        pltpu.make_async_copy(rows[b], acc_sh.at[dst_v.at[0]], ssem[b]).wait()

    for b in range(NBA):
        sidx_start(b, b)
        didx_start(b, b)
    for b in range(NBA):
        sidx_wait(b)
        gather_start(b)

    def outer(o, carry):
        for b in range(NBA):
            i = o * NBA + b
            gather_wait(b)
            didx_wait(b)
            scatter_start(b)

            @pl.when(i + NBA < NB)
            def _():
                sidx_start(i + NBA, b)

        for b in range(NBA):
            nxt = (o + 1) * NBA + b
            scatter_wait(b)

            @pl.when(nxt < NB)
            def _():
                didx_start(nxt, b)
                sidx_wait(b)
                gather_start(b)

        return carry

    lax.fori_loop(0, NB // NBA, outer, 0)
    for t in range((NB // NBA) * NBA, NB):
        b = t % NBA
        gather_wait(b)
        didx_wait(b)
        scatter_start(b)
        scatter_wait(b)
    plsc.subcore_barrier()
    _acc_spill(acc_sh, out_hbm, c, s)


BR = 1000


def _dis(degp_ref):
    deg = degp_ref[0, :, 0:1] + degp_ref[1, :, 0:1] + 1.0
    return lax.rsqrt(deg)


def _k1_body(x_ref, w_ref, degp_ref, o_ref):
    dis = _dis(degp_ref)
    o_ref[...] = dis * jnp.dot(x_ref[...], w_ref[...],
                               preferred_element_type=jnp.float32)


def _k2_body(p_ref, g_ref, b_ref, degp_ref, w_ref, h_ref, g2_ref):
    dis = _dis(degp_ref)
    h = jnp.maximum(dis * (p_ref[0] + p_ref[1] + g_ref[...]) + b_ref[...], 0.0)
    h_ref[...] = h
    g2_ref[...] = dis * jnp.dot(h, w_ref[...],
                                preferred_element_type=jnp.float32)


def _k3_body(p_ref, g_ref, b_ref, degp_ref, h1_ref, wfc_ref, bfc_ref,
             emb_ref, log_ref):
    dis = _dis(degp_ref)
    h2 = jnp.maximum(dis * (p_ref[0] + p_ref[1] + g_ref[...]) + b_ref[...], 0.0)
    emb = jnp.maximum(h1_ref[...], h2)
    emb_ref[...] = emb
    log_ref[...] = jnp.dot(emb, wfc_ref[...],
                           preferred_element_type=jnp.float32) + bfc_ref[...]


def _row_spec(width):
    return pl.BlockSpec((BR, width), lambda i: (i, 0))


def _part_spec(width):
    return pl.BlockSpec((NC, BR, width), lambda i: (0, i, 0))


def _full_spec(shape):
    return pl.BlockSpec(shape, lambda i: tuple(0 for _ in shape))


def kernel(x, edge_index, W1, b1, W2, b2, Wfc, bfc):
    src = edge_index[0].astype(jnp.int32)
    dstf = edge_index[1].astype(jnp.int32)
    dst = dstf.reshape(NW, NB, BE)
    zF = jnp.zeros((N, F), jnp.float32)
    ones = jnp.ones((BE, F), jnp.float32)

    degp = _deg_kernel(dst, zF, ones)[:, :, :8]

    g1 = pl.pallas_call(
        _k1_body,
        grid=(N // BR,),
        in_specs=[_row_spec(F), _full_spec((F, F)), _part_spec(8)],
        out_specs=_row_spec(F),
        out_shape=jax.ShapeDtypeStruct((N, F), jnp.float32),
    )(x, W1, degp)

    p1 = _agg_kernel(g1, src, dstf, zF)

    h1, g2 = pl.pallas_call(
        _k2_body,
        grid=(N // BR,),
        in_specs=[_part_spec(F), _row_spec(F), _full_spec((1, F)),
                  _part_spec(8), _full_spec((F, F))],
        out_specs=[_row_spec(F), _row_spec(F)],
        out_shape=[jax.ShapeDtypeStruct((N, F), jnp.float32),
                   jax.ShapeDtypeStruct((N, F), jnp.float32)],
    )(p1, g1, b1.reshape(1, F), degp, W2)

    p2 = _agg_kernel(g2, src, dstf, zF)

    emb, logits = pl.pallas_call(
        _k3_body,
        grid=(N // BR,),
        in_specs=[_part_spec(F), _row_spec(F), _full_spec((1, F)),
                  _part_spec(8), _row_spec(F), _full_spec((F, NCLASS)),
                  _full_spec((1, NCLASS))],
        out_specs=[_row_spec(F), _row_spec(NCLASS)],
        out_shape=[jax.ShapeDtypeStruct((N, F), jnp.float32),
                   jax.ShapeDtypeStruct((N, NCLASS), jnp.float32)],
    )(p2, g2, b2.reshape(1, F), degp, h1, Wfc, bfc.reshape(1, NCLASS))

    return emb, logits
